# Initial kernel scaffold; baseline (speedup 1.0000x reference)
import jax
import jax.numpy as jnp
from jax.experimental import pallas as pl


def kernel(x, table):
    raise NotImplementedError("write your pallas kernel here")


# speedup vs baseline: 1.2227x; 1.2227x over previous
import functools

import jax
import jax.numpy as jnp
from jax import lax
from jax.experimental import pallas as pl
from jax.experimental.pallas import tpu as pltpu
from jax.experimental.pallas import tpu_sc as plsc

_VOCAB = 512
_BATCH = 1024
_SEQ = 200
_ROWS = _BATCH * (_SEQ - 1)
_NC, _NS = 2, 16
_NW = _NC * _NS
_CHUNK = 128
_NCHUNKS = _ROWS // _CHUNK
_ITERS = -(-_NCHUNKS // _NW)


def _make_sc_gather():
    mesh = plsc.VectorSubcoreMesh(
        core_axis_name="c", subcore_axis_name="s",
        num_cores=_NC, num_subcores=_NS)

    @functools.partial(
        pl.kernel,
        out_type=jax.ShapeDtypeStruct((_ROWS, _VOCAB), jnp.float32),
        mesh=mesh,
        scratch_types=[
            pltpu.VMEM((_CHUNK,), jnp.int32),
            pltpu.VMEM((_CHUNK,), jnp.int32),
            pltpu.VMEM((_CHUNK,), jnp.int32),
            pltpu.VMEM((_CHUNK, _VOCAB), jnp.float32),
            pltpu.SemaphoreType.DMA,
        ],
    )
    def sc_gather(a_hbm, b_hbm, table_hbm, out_hbm, a_v, b_v, idx_v, rows_v,
                  sem):
        wid = lax.axis_index("s") * _NC + lax.axis_index("c")

        def body(i, carry):
            c = i * _NW + wid

            @pl.when(c < _NCHUNKS)
            def _():
                off = c * _CHUNK
                pltpu.sync_copy(a_hbm.at[pl.ds(off, _CHUNK)], a_v)
                pltpu.sync_copy(b_hbm.at[pl.ds(off, _CHUNK)], b_v)
                for j in range(_CHUNK // 16):
                    s = pl.ds(j * 16, 16)
                    idx_v[s] = a_v[s] * _VOCAB + b_v[s]
                pltpu.async_copy(table_hbm.at[idx_v], rows_v, sem).wait()
                pltpu.sync_copy(rows_v, out_hbm.at[pl.ds(off, _CHUNK)])

            return carry

        lax.fori_loop(0, _ITERS, body, 0)

    return sc_gather


_sc_gather = _make_sc_gather()


def kernel(x, table):
    a = x[:, :-1].reshape(-1)
    b = x[:, 1:].reshape(-1)
    logits = _sc_gather(a, b, table)
    return logits.reshape(_BATCH, _SEQ - 1, _VOCAB)


# speedup vs baseline: 1.3074x; 1.0692x over previous
import functools

import jax
import jax.numpy as jnp
from jax import lax
from jax.experimental import pallas as pl
from jax.experimental.pallas import tpu as pltpu
from jax.experimental.pallas import tpu_sc as plsc

_VOCAB = 512
_BATCH = 1024
_SEQ = 200
_ROWS = _BATCH * (_SEQ - 1)
_NC, _NS = 2, 16
_NW = _NC * _NS
_PER_W = _ROWS // _NW
_CHUNK = 32
_NCHUNK = _PER_W // _CHUNK
_NBUF = 4
_NGROUP = -(-_NCHUNK // _NBUF)


def _make_sc_gather():
    mesh = plsc.VectorSubcoreMesh(
        core_axis_name="c", subcore_axis_name="s",
        num_cores=_NC, num_subcores=_NS)

    @functools.partial(
        pl.kernel,
        out_type=jax.ShapeDtypeStruct((_ROWS, _VOCAB), jnp.float32),
        mesh=mesh,
        scratch_types=[
            pltpu.VMEM((_PER_W,), jnp.int32),
            pltpu.VMEM((_PER_W,), jnp.int32),
            pltpu.VMEM((_PER_W,), jnp.int32),
        ] + [pltpu.VMEM((_CHUNK, _VOCAB), jnp.float32)] * _NBUF
          + [pltpu.SemaphoreType.DMA] * (2 * _NBUF),
    )
    def sc_gather(a_hbm, b_hbm, table_hbm, out_hbm, a_v, b_v, idx_v, *rest):
        bufs = rest[:_NBUF]
        gsems = rest[_NBUF:2 * _NBUF]
        ssems = rest[2 * _NBUF:]

        wid = lax.axis_index("s") * _NC + lax.axis_index("c")
        base = pl.multiple_of(wid * _PER_W, _PER_W)
        pltpu.sync_copy(a_hbm.at[pl.ds(base, _PER_W)], a_v)
        pltpu.sync_copy(b_hbm.at[pl.ds(base, _PER_W)], b_v)

        def idx_body(i, carry):
            s = pl.ds(pl.multiple_of(i * 16, 16), 16)
            idx_v[s] = a_v[s] * _VOCAB + b_v[s]
            return carry

        lax.fori_loop(0, _PER_W // 16, idx_body, 0)

        def gather(c, b):
            off = pl.multiple_of(c * _CHUNK, _CHUNK)
            return pltpu.async_copy(
                table_hbm.at[idx_v.at[pl.ds(off, _CHUNK)]], bufs[b], gsems[b])

        def scatter(c, b):
            off = pl.multiple_of(base + c * _CHUNK, _CHUNK)
            return pltpu.async_copy(
                bufs[b], out_hbm.at[pl.ds(off, _CHUNK)], ssems[b])

        def group(g, carry):
            for b in range(_NBUF):
                c = g * _NBUF + b

                @pl.when(c < _NCHUNK)
                def _(c=c, b=b):
                    @pl.when(g > 0)
                    def _():
                        off = pl.multiple_of(base + (c - _NBUF) * _CHUNK,
                                             _CHUNK)
                        pltpu.make_async_copy(
                            bufs[b], out_hbm.at[pl.ds(off, _CHUNK)],
                            ssems[b]).wait()

                    gather(c, b)

            for b in range(_NBUF):
                c = g * _NBUF + b

                @pl.when(c < _NCHUNK)
                def _(c=c, b=b):
                    off = pl.multiple_of(c * _CHUNK, _CHUNK)
                    pltpu.make_async_copy(
                        table_hbm.at[idx_v.at[pl.ds(off, _CHUNK)]], bufs[b],
                        gsems[b]).wait()
                    scatter(c, b)

            return carry

        lax.fori_loop(0, _NGROUP, group, 0)

        for b in range(_NBUF):
            pltpu.make_async_copy(
                bufs[b], out_hbm.at[pl.ds(base, _CHUNK)], ssems[b]).wait()

    return sc_gather


_sc_gather = _make_sc_gather()


def kernel(x, table):
    a = x[:, :-1].reshape(-1)
    b = x[:, 1:].reshape(-1)
    logits = _sc_gather(a, b, table)
    return logits.reshape(_BATCH, _SEQ - 1, _VOCAB)


# speedup vs baseline: 3.6179x; 2.7673x over previous
import functools

import jax
import jax.numpy as jnp
from jax import lax
from jax.experimental import pallas as pl
from jax.experimental.pallas import tpu as pltpu
from jax.experimental.pallas import tpu_sc as plsc

_VOCAB = 512
_BATCH = 1024
_SEQ = 200
_ROWS = _BATCH * (_SEQ - 1)
_NC, _NS = 2, 16
_NW = _NC * _NS
_PER_W = _ROWS // _NW
_CHUNK = 32
_NCHUNK = _PER_W // _CHUNK
_NBUF = 4
_NGROUP = -(-_NCHUNK // _NBUF)


def _make_sc_gather():
    mesh = plsc.VectorSubcoreMesh(
        core_axis_name="c", subcore_axis_name="s",
        num_cores=_NC, num_subcores=_NS)

    @functools.partial(
        pl.kernel,
        out_type=jax.ShapeDtypeStruct((_ROWS, _VOCAB), jnp.float32),
        mesh=mesh,
        scratch_types=[
            pltpu.VMEM((_PER_W,), jnp.int32),
            pltpu.VMEM((_PER_W,), jnp.int32),
            pltpu.VMEM((_PER_W,), jnp.int32),
        ] + [pltpu.VMEM((_CHUNK, _VOCAB), jnp.float32)] * _NBUF
          + [pltpu.SemaphoreType.DMA] * (2 * _NBUF),
    )
    def sc_gather(a_hbm, b_hbm, table_hbm, out_hbm, a_v, b_v, idx_v, *rest):
        bufs = rest[:_NBUF]
        gsems = rest[_NBUF:2 * _NBUF]
        ssems = rest[2 * _NBUF:]

        wid = lax.axis_index("s") * _NC + lax.axis_index("c")
        base = pl.multiple_of(wid * _PER_W, _PER_W)
        pltpu.sync_copy(a_hbm.at[pl.ds(base, _PER_W)], a_v)
        pltpu.sync_copy(b_hbm.at[pl.ds(base, _PER_W)], b_v)

        def idx_body(i, carry):
            s = pl.ds(pl.multiple_of(i * 16, 16), 16)
            idx_v[s] = a_v[s] * _VOCAB + b_v[s]
            return carry

        lax.fori_loop(0, _PER_W // 16, idx_body, 0)

        def gather(c, b):
            off = pl.multiple_of(c * _CHUNK, _CHUNK)
            return pltpu.async_copy(
                table_hbm.at[idx_v.at[pl.ds(off, _CHUNK)]], bufs[b], gsems[b])

        def scatter(c, b):
            off = pl.multiple_of(base + c * _CHUNK, _CHUNK)
            return pltpu.async_copy(
                bufs[b], out_hbm.at[pl.ds(off, _CHUNK)], ssems[b])

        def group(g, carry):
            for b in range(_NBUF):
                c = g * _NBUF + b

                @pl.when(c < _NCHUNK)
                def _(c=c, b=b):
                    @pl.when(g > 0)
                    def _():
                        off = pl.multiple_of(base + (c - _NBUF) * _CHUNK,
                                             _CHUNK)
                        pltpu.make_async_copy(
                            bufs[b], out_hbm.at[pl.ds(off, _CHUNK)],
                            ssems[b]).wait()

                    gather(c, b)

            for b in range(_NBUF):
                c = g * _NBUF + b

                @pl.when(c < _NCHUNK)
                def _(c=c, b=b):
                    off = pl.multiple_of(c * _CHUNK, _CHUNK)
                    pltpu.make_async_copy(
                        table_hbm.at[idx_v.at[pl.ds(off, _CHUNK)]], bufs[b],
                        gsems[b]).wait()
                    scatter(c, b)

            return carry

        lax.fori_loop(0, _NGROUP, group, 0)

        for b in range(_NBUF):
            pltpu.make_async_copy(
                bufs[b], out_hbm.at[pl.ds(base, _CHUNK)], ssems[b]).wait()

    return sc_gather


_sc_gather = _make_sc_gather()


def kernel(x, table):
    a = x[:, :-1].T.reshape(-1)
    b = x[:, 1:].T.reshape(-1)
    logits = _sc_gather(a, b, table)
    return jnp.transpose(
        logits.reshape(_SEQ - 1, _BATCH, _VOCAB), (1, 0, 2))


# speedup vs baseline: 3.6230x; 1.0014x over previous
import functools

import jax
import jax.numpy as jnp
from jax import lax
from jax.experimental import pallas as pl
from jax.experimental.pallas import tpu as pltpu
from jax.experimental.pallas import tpu_sc as plsc

_VOCAB = 512
_BATCH = 1024
_SEQ = 200
_ROWS = _BATCH * (_SEQ - 1)
_NC, _NS = 2, 16
_NW = _NC * _NS
_PER_W = _ROWS // _NW
_CHUNK = 64
_NCHUNK = _PER_W // _CHUNK
_TAIL = _PER_W - _NCHUNK * _CHUNK
_NBUF = 3
_NGROUP = _NCHUNK // _NBUF


def _make_sc_gather():
    mesh = plsc.VectorSubcoreMesh(
        core_axis_name="c", subcore_axis_name="s",
        num_cores=_NC, num_subcores=_NS)

    @functools.partial(
        pl.kernel,
        out_type=jax.ShapeDtypeStruct((_ROWS, _VOCAB), jnp.float32),
        mesh=mesh,
        scratch_types=[
            pltpu.VMEM((_PER_W,), jnp.int32),
            pltpu.VMEM((_PER_W,), jnp.int32),
            pltpu.VMEM((_PER_W,), jnp.int32),
        ] + [pltpu.VMEM((_CHUNK, _VOCAB), jnp.float32)] * _NBUF
          + [pltpu.SemaphoreType.DMA] * (2 * _NBUF),
    )
    def sc_gather(a_hbm, b_hbm, table_hbm, out_hbm, a_v, b_v, idx_v, *rest):
        bufs = rest[:_NBUF]
        gsems = rest[_NBUF:2 * _NBUF]
        ssems = rest[2 * _NBUF:]

        wid = lax.axis_index("s") * _NC + lax.axis_index("c")
        base = pl.multiple_of(wid * _PER_W, _PER_W)
        pltpu.sync_copy(a_hbm.at[pl.ds(base, _PER_W)], a_v)
        pltpu.sync_copy(b_hbm.at[pl.ds(base, _PER_W)], b_v)

        def idx_body(i, carry):
            s = pl.ds(pl.multiple_of(i * 16, 16), 16)
            idx_v[s] = a_v[s] * _VOCAB + b_v[s]
            return carry

        lax.fori_loop(0, _PER_W // 16, idx_body, 0)

        def gather(c, b):
            off = pl.multiple_of(c * _CHUNK, _CHUNK)
            return pltpu.async_copy(
                table_hbm.at[idx_v.at[pl.ds(off, _CHUNK)]], bufs[b], gsems[b])

        def scatter(c, b):
            off = pl.multiple_of(base + c * _CHUNK, _CHUNK)
            return pltpu.async_copy(
                bufs[b], out_hbm.at[pl.ds(off, _CHUNK)], ssems[b])

        def group(g, carry):
            for b in range(_NBUF):
                c = g * _NBUF + b

                @pl.when(g > 0)
                def _(c=c, b=b):
                    off = pl.multiple_of(base + (c - _NBUF) * _CHUNK, _CHUNK)
                    pltpu.make_async_copy(
                        bufs[b], out_hbm.at[pl.ds(off, _CHUNK)],
                        ssems[b]).wait()

                gather(c, b)

            for b in range(_NBUF):
                c = g * _NBUF + b
                off = pl.multiple_of(c * _CHUNK, _CHUNK)
                pltpu.make_async_copy(
                    table_hbm.at[idx_v.at[pl.ds(off, _CHUNK)]], bufs[b],
                    gsems[b]).wait()
                scatter(c, b)

            return carry

        lax.fori_loop(0, _NGROUP, group, 0)

        tail_off = pl.multiple_of(_NCHUNK * _CHUNK, _CHUNK)
        pltpu.make_async_copy(
            bufs[0], out_hbm.at[pl.ds(base, _CHUNK)], ssems[0]).wait()
        pltpu.async_copy(
            table_hbm.at[idx_v.at[pl.ds(tail_off, _TAIL)]],
            bufs[0].at[pl.ds(0, _TAIL)], gsems[0]).wait()
        pltpu.sync_copy(bufs[0].at[pl.ds(0, _TAIL)],
                        out_hbm.at[pl.ds(base + tail_off, _TAIL)])

        for b in range(1, _NBUF):
            pltpu.make_async_copy(
                bufs[b], out_hbm.at[pl.ds(base, _CHUNK)], ssems[b]).wait()

    return sc_gather


_sc_gather = _make_sc_gather()


def kernel(x, table):
    a = x[:, :-1].T.reshape(-1)
    b = x[:, 1:].T.reshape(-1)
    logits = _sc_gather(a, b, table)
    return jnp.transpose(
        logits.reshape(_SEQ - 1, _BATCH, _VOCAB), (1, 0, 2))
